# initial kernel scaffold (unmeasured)
import functools

import jax
import jax.numpy as jnp
from jax import lax
from jax.experimental import pallas as pl
from jax.experimental.pallas import tpu as pltpu

M = 4096
D = 4096
HALF = M // 2
CH = 256
NC = HALF // CH
EPS = 1e-6


def kernel(partial, resid, gamma):
    partial2d = partial.reshape(M, D)
    gamma2d = gamma.reshape(1, D)

    def body(
        partial_ref, resid_ref, gamma_ref, out_ref,
        sendA, recvA, sendB, recvB,
        pA, res_v, rA_v, o_v, rB_v, oB_v,
        loc_sems, sendA_sems, recvA_sems, sendB_sems, recvB_sems,
        outA_sems, outB_sems,
    ):
        my_x = lax.axis_index("x")
        my_y = lax.axis_index("y")
        y_nbr = (my_x, 1 - my_y)
        x_nbr = (1 - my_x, my_y)

        barrier_sem = pltpu.get_barrier_semaphore()
        for nbr in (y_nbr, x_nbr):
            pl.semaphore_signal(
                barrier_sem, inc=1, device_id=nbr,
                device_id_type=pl.DeviceIdType.MESH,
            )
        pl.semaphore_wait(barrier_sem, 2)

        half_start = my_x * HALF
        other_start = HALF - half_start

        def recvA_desc(c):
            return pltpu.make_async_remote_copy(
                src_ref=sendA.at[0],
                dst_ref=recvA.at[pl.ds(c * CH, CH), :],
                send_sem=sendA_sems.at[0],
                recv_sem=recvA_sems.at[c],
                device_id=y_nbr,
                device_id_type=pl.DeviceIdType.MESH,
            )

        def recvB_desc(c):
            return pltpu.make_async_remote_copy(
                src_ref=sendB.at[0],
                dst_ref=recvB.at[pl.ds(c * CH, CH), :],
                send_sem=sendB_sems.at[0],
                recv_sem=recvB_sems.at[c],
                device_id=x_nbr,
                device_id_type=pl.DeviceIdType.MESH,
            )

        rdmaA = []
        for c in range(NC):
            slot = c % 2
            rows = pl.ds(half_start + c * CH, CH)
            cp = pltpu.make_async_copy(
                partial_ref.at[rows, :], pA.at[slot], loc_sems.at[slot]
            )
            cp.start()
            cp.wait()
            if c >= 2:
                rdmaA[c - 2].wait_send()
            sendA[slot, :, :] = pA[slot, :, :].astype(jnp.bfloat16)
            r = pltpu.make_async_remote_copy(
                src_ref=sendA.at[slot],
                dst_ref=recvA.at[pl.ds(c * CH, CH), :],
                send_sem=sendA_sems.at[slot],
                recv_sem=recvA_sems.at[c],
                device_id=y_nbr,
                device_id_type=pl.DeviceIdType.MESH,
            )
            r.start()
            rdmaA.append(r)
        rdmaA[NC - 2].wait_send()
        rdmaA[NC - 1].wait_send()

        rdmaB = []
        outA = []
        for c in range(NC):
            slot = c % 2
            rows = pl.ds(half_start + c * CH, CH)
            recvA_desc(c).wait_recv()
            cp0 = pltpu.make_async_copy(
                recvA.at[pl.ds(c * CH, CH), :], rA_v.at[slot], loc_sems.at[0]
            )
            cp1 = pltpu.make_async_copy(
                partial_ref.at[rows, :], pA.at[slot], loc_sems.at[1]
            )
            cp2 = pltpu.make_async_copy(
                resid_ref.at[rows, :], res_v.at[slot], loc_sems.at[2]
            )
            cp0.start(); cp1.start(); cp2.start()
            cp0.wait(); cp1.wait(); cp2.wait()
            if c >= 2:
                outA[c - 2].wait()
                rdmaB[c - 2].wait_send()
            y = (
                pA[slot, :, :]
                + rA_v[slot, :, :].astype(jnp.float32)
                + res_v[slot, :, :]
            )
            inv = lax.rsqrt(jnp.sum(y * y, axis=-1, keepdims=True) / D + EPS)
            o = y * inv * gamma_ref[:, :]
            o_v[slot, :, :] = o
            sendB[slot, :, :] = o.astype(jnp.bfloat16)
            od = pltpu.make_async_copy(
                o_v.at[slot], out_ref.at[rows, :], outA_sems.at[slot]
            )
            od.start()
            outA.append(od)
            rb = pltpu.make_async_remote_copy(
                src_ref=sendB.at[slot],
                dst_ref=recvB.at[pl.ds(c * CH, CH), :],
                send_sem=sendB_sems.at[slot],
                recv_sem=recvB_sems.at[c],
                device_id=x_nbr,
                device_id_type=pl.DeviceIdType.MESH,
            )
            rb.start()
            rdmaB.append(rb)
        for c in (NC - 2, NC - 1):
            outA[c].wait()
            rdmaB[c].wait_send()

        outB = []
        for c in range(NC):
            slot = c % 2
            orows = pl.ds(other_start + c * CH, CH)
            recvB_desc(c).wait_recv()
            cp = pltpu.make_async_copy(
                recvB.at[pl.ds(c * CH, CH), :], rB_v.at[slot], loc_sems.at[3]
            )
            cp.start()
            cp.wait()
            if c >= 2:
                outB[c - 2].wait()
            oB_v[slot, :, :] = rB_v[slot, :, :].astype(jnp.float32)
            od = pltpu.make_async_copy(
                oB_v.at[slot], out_ref.at[orows, :], outB_sems.at[slot]
            )
            od.start()
            outB.append(od)
        for c in (NC - 2, NC - 1):
            outB[c].wait()

        @functools.partial(pl.run_scoped, sem=pltpu.SemaphoreType.REGULAR)
        def _(sem):
            for nbr in (y_nbr, x_nbr):
                pl.semaphore_signal(
                    sem, inc=1, device_id=nbr,
                    device_id_type=pl.DeviceIdType.MESH,
                )
            pl.semaphore_wait(sem, 2)

    return pl.pallas_call(
        body,
        out_shape=jax.ShapeDtypeStruct((M, D), jnp.float32),
        in_specs=[
            pl.BlockSpec(memory_space=pltpu.MemorySpace.HBM),
            pl.BlockSpec(memory_space=pltpu.MemorySpace.HBM),
            pl.BlockSpec(memory_space=pltpu.MemorySpace.VMEM),
        ],
        out_specs=pl.BlockSpec(memory_space=pltpu.MemorySpace.HBM),
        scratch_shapes=[
            pltpu.MemorySpace.VMEM((2, CH, D), jnp.bfloat16),
            pltpu.MemorySpace.HBM((HALF, D), jnp.bfloat16),
            pltpu.MemorySpace.VMEM((2, CH, D), jnp.bfloat16),
            pltpu.MemorySpace.HBM((HALF, D), jnp.bfloat16),
            pltpu.MemorySpace.VMEM((2, CH, D), jnp.float32),
            pltpu.MemorySpace.VMEM((2, CH, D), jnp.float32),
            pltpu.MemorySpace.VMEM((2, CH, D), jnp.bfloat16),
            pltpu.MemorySpace.VMEM((2, CH, D), jnp.float32),
            pltpu.MemorySpace.VMEM((2, CH, D), jnp.bfloat16),
            pltpu.MemorySpace.VMEM((2, CH, D), jnp.float32),
            pltpu.SemaphoreType.DMA((4,)),
            pltpu.SemaphoreType.DMA((2,)),
            pltpu.SemaphoreType.DMA((NC,)),
            pltpu.SemaphoreType.DMA((2,)),
            pltpu.SemaphoreType.DMA((NC,)),
            pltpu.SemaphoreType.DMA((2,)),
            pltpu.SemaphoreType.DMA((2,)),
        ],
        compiler_params=pltpu.CompilerParams(collective_id=0),
    )(partial2d, resid, gamma2d)


# baseline (device time: 425958 ns/iter reference)
import functools

import jax
import jax.numpy as jnp
from jax import lax
from jax.experimental import pallas as pl
from jax.experimental.pallas import tpu as pltpu

M = 4096
D = 4096
HALF = M // 2
CH = 128
NC = HALF // CH
EPS = 1e-6


def kernel(partial, resid, gamma):
    partial2d = partial.reshape(M, D)
    gamma2d = gamma.reshape(1, D)

    def body(
        partial_ref, resid_ref, gamma_ref, out_ref,
        sendA, recvA, sendB, recvB,
        pA, res_v, o_v, oB_v,
        loc_sems, sendA_sems, recvA_sems, sendB_sems, recvB_sems,
        outA_sems, outB_sems,
    ):
        my_x = lax.axis_index("x")
        my_y = lax.axis_index("y")
        y_nbr = (my_x, 1 - my_y)
        x_nbr = (1 - my_x, my_y)

        barrier_sem = pltpu.get_barrier_semaphore()
        for nbr in (y_nbr, x_nbr):
            pl.semaphore_signal(
                barrier_sem, inc=1, device_id=nbr,
                device_id_type=pl.DeviceIdType.MESH,
            )
        pl.semaphore_wait(barrier_sem, 2)

        half_start = my_x * HALF
        other_start = HALF - half_start

        def recvA_desc(c):
            return pltpu.make_async_remote_copy(
                src_ref=sendA.at[0],
                dst_ref=recvA.at[pl.ds(c * CH, CH), :],
                send_sem=sendA_sems.at[0],
                recv_sem=recvA_sems.at[c],
                device_id=y_nbr,
                device_id_type=pl.DeviceIdType.MESH,
            )

        def recvB_desc(c):
            return pltpu.make_async_remote_copy(
                src_ref=sendB.at[0],
                dst_ref=recvB.at[pl.ds(c * CH, CH), :],
                send_sem=sendB_sems.at[0],
                recv_sem=recvB_sems.at[c],
                device_id=x_nbr,
                device_id_type=pl.DeviceIdType.MESH,
            )

        rdmaA = []
        for c in range(NC):
            slot = c % 2
            rows = pl.ds(half_start + c * CH, CH)
            cp = pltpu.make_async_copy(
                partial_ref.at[rows, :], pA.at[slot], loc_sems.at[slot]
            )
            cp.start()
            cp.wait()
            if c >= 2:
                rdmaA[c - 2].wait_send()
            sendA[slot, :, :] = pA[slot, :, :].astype(jnp.bfloat16)
            r = pltpu.make_async_remote_copy(
                src_ref=sendA.at[slot],
                dst_ref=recvA.at[pl.ds(c * CH, CH), :],
                send_sem=sendA_sems.at[slot],
                recv_sem=recvA_sems.at[c],
                device_id=y_nbr,
                device_id_type=pl.DeviceIdType.MESH,
            )
            r.start()
            rdmaA.append(r)
        rdmaA[NC - 2].wait_send()
        rdmaA[NC - 1].wait_send()

        rdmaB = []
        outA = []
        for c in range(NC):
            slot = c % 2
            rows = pl.ds(half_start + c * CH, CH)
            cp1 = pltpu.make_async_copy(
                partial_ref.at[rows, :], pA.at[slot], loc_sems.at[1]
            )
            cp2 = pltpu.make_async_copy(
                resid_ref.at[rows, :], res_v.at[slot], loc_sems.at[2]
            )
            cp1.start(); cp2.start()
            recvA_desc(c).wait_recv()
            cp1.wait(); cp2.wait()
            if c >= 2:
                outA[c - 2].wait()
                rdmaB[c - 2].wait_send()
            y = (
                pA[slot, :, :]
                + recvA[c * CH:(c + 1) * CH, :].astype(jnp.float32)
                + res_v[slot, :, :]
            )
            inv = lax.rsqrt(jnp.sum(y * y, axis=-1, keepdims=True) / D + EPS)
            o = y * inv * gamma_ref[:, :]
            o_v[slot, :, :] = o
            sendB[slot, :, :] = o.astype(jnp.bfloat16)
            od = pltpu.make_async_copy(
                o_v.at[slot], out_ref.at[rows, :], outA_sems.at[slot]
            )
            od.start()
            outA.append(od)
            rb = pltpu.make_async_remote_copy(
                src_ref=sendB.at[slot],
                dst_ref=recvB.at[pl.ds(c * CH, CH), :],
                send_sem=sendB_sems.at[slot],
                recv_sem=recvB_sems.at[c],
                device_id=x_nbr,
                device_id_type=pl.DeviceIdType.MESH,
            )
            rb.start()
            rdmaB.append(rb)
        for c in (NC - 2, NC - 1):
            outA[c].wait()
            rdmaB[c].wait_send()

        outB = []
        for c in range(NC):
            slot = c % 2
            orows = pl.ds(other_start + c * CH, CH)
            recvB_desc(c).wait_recv()
            if c >= 2:
                outB[c - 2].wait()
            oB_v[slot, :, :] = recvB[c * CH:(c + 1) * CH, :].astype(jnp.float32)
            od = pltpu.make_async_copy(
                oB_v.at[slot], out_ref.at[orows, :], outB_sems.at[slot]
            )
            od.start()
            outB.append(od)
        for c in (NC - 2, NC - 1):
            outB[c].wait()

        @functools.partial(pl.run_scoped, sem=pltpu.SemaphoreType.REGULAR)
        def _(sem):
            for nbr in (y_nbr, x_nbr):
                pl.semaphore_signal(
                    sem, inc=1, device_id=nbr,
                    device_id_type=pl.DeviceIdType.MESH,
                )
            pl.semaphore_wait(sem, 2)

    return pl.pallas_call(
        body,
        out_shape=jax.ShapeDtypeStruct((M, D), jnp.float32),
        in_specs=[
            pl.BlockSpec(memory_space=pltpu.MemorySpace.HBM),
            pl.BlockSpec(memory_space=pltpu.MemorySpace.HBM),
            pl.BlockSpec(memory_space=pltpu.MemorySpace.VMEM),
        ],
        out_specs=pl.BlockSpec(memory_space=pltpu.MemorySpace.HBM),
        scratch_shapes=[
            pltpu.MemorySpace.VMEM((2, CH, D), jnp.bfloat16),
            pltpu.MemorySpace.VMEM((HALF, D), jnp.bfloat16),
            pltpu.MemorySpace.VMEM((2, CH, D), jnp.bfloat16),
            pltpu.MemorySpace.VMEM((HALF, D), jnp.bfloat16),
            pltpu.MemorySpace.VMEM((2, CH, D), jnp.float32),
            pltpu.MemorySpace.VMEM((2, CH, D), jnp.float32),
            pltpu.MemorySpace.VMEM((2, CH, D), jnp.float32),
            pltpu.MemorySpace.VMEM((2, CH, D), jnp.float32),
            pltpu.SemaphoreType.DMA((4,)),
            pltpu.SemaphoreType.DMA((2,)),
            pltpu.SemaphoreType.DMA((NC,)),
            pltpu.SemaphoreType.DMA((2,)),
            pltpu.SemaphoreType.DMA((NC,)),
            pltpu.SemaphoreType.DMA((2,)),
            pltpu.SemaphoreType.DMA((2,)),
        ],
        compiler_params=pltpu.CompilerParams(
            collective_id=0, vmem_limit_bytes=62 * 1024 * 1024
        ),
    )(partial2d, resid, gamma2d)


# device time: 329050 ns/iter; 1.2945x vs baseline; 1.2945x over previous
import functools

import jax
import jax.numpy as jnp
from jax import lax
from jax.experimental import pallas as pl
from jax.experimental.pallas import tpu as pltpu

M = 4096
D = 4096
HALF = M // 2
CH = 128
NC = HALF // CH
EPS = 1e-6


def kernel(partial, resid, gamma):
    partial2d = partial.reshape(M, D)
    gamma2d = gamma.reshape(1, D)

    def body(
        partial_ref, resid_ref, gamma_ref, out_ref,
        sendA, recvA, sendB, recvB,
        pA, res_v, o_v, oB_v,
        loc_sems, sendA_sems, recvA_sems, sendB_sems, recvB_sems,
        outA_sems, outB_sems,
    ):
        my_x = lax.axis_index("x")
        my_y = lax.axis_index("y")
        y_nbr = (my_x, 1 - my_y)
        x_nbr = (1 - my_x, my_y)

        barrier_sem = pltpu.get_barrier_semaphore()
        for nbr in (y_nbr, x_nbr):
            pl.semaphore_signal(
                barrier_sem, inc=1, device_id=nbr,
                device_id_type=pl.DeviceIdType.MESH,
            )
        pl.semaphore_wait(barrier_sem, 2)

        half_start = my_x * HALF
        other_start = HALF - half_start

        def recvA_desc(c):
            return pltpu.make_async_remote_copy(
                src_ref=sendA.at[0],
                dst_ref=recvA.at[pl.ds(c * CH, CH), :],
                send_sem=sendA_sems.at[0],
                recv_sem=recvA_sems.at[c],
                device_id=y_nbr,
                device_id_type=pl.DeviceIdType.MESH,
            )

        def recvB_desc(c):
            return pltpu.make_async_remote_copy(
                src_ref=sendB.at[0],
                dst_ref=recvB.at[pl.ds(c * CH, CH), :],
                send_sem=sendB_sems.at[0],
                recv_sem=recvB_sems.at[c],
                device_id=x_nbr,
                device_id_type=pl.DeviceIdType.MESH,
            )

        LAG = 2
        rdmaA, rdmaB, outA, outB = [], [], [], []

        def store_other_half(d):
            dslot = d % 2
            orows = pl.ds(other_start + d * CH, CH)
            recvB_desc(d).wait_recv()
            if d >= 2:
                outB[d - 2].wait()
            oB_v[dslot, :, :] = recvB[d * CH:(d + 1) * CH, :].astype(
                jnp.float32
            )
            od = pltpu.make_async_copy(
                oB_v.at[dslot], out_ref.at[orows, :], outB_sems.at[dslot]
            )
            od.start()
            outB.append(od)

        for c in range(NC):
            slot = c % 2
            rows = pl.ds(half_start + c * CH, CH)
            if c >= 2:
                rdmaA[c - 2].wait_send()
                rdmaB[c - 2].wait_send()
                outA[c - 2].wait()
            cp1 = pltpu.make_async_copy(
                partial_ref.at[rows, :], pA.at[slot], loc_sems.at[0]
            )
            cp2 = pltpu.make_async_copy(
                resid_ref.at[rows, :], res_v.at[slot], loc_sems.at[1]
            )
            cp1.start(); cp2.start()
            cp1.wait()
            sendA[slot, :, :] = pA[slot, :, :].astype(jnp.bfloat16)
            r = pltpu.make_async_remote_copy(
                src_ref=sendA.at[slot],
                dst_ref=recvA.at[pl.ds(c * CH, CH), :],
                send_sem=sendA_sems.at[slot],
                recv_sem=recvA_sems.at[c],
                device_id=y_nbr,
                device_id_type=pl.DeviceIdType.MESH,
            )
            r.start()
            rdmaA.append(r)
            recvA_desc(c).wait_recv()
            cp2.wait()
            y = (
                pA[slot, :, :]
                + recvA[c * CH:(c + 1) * CH, :].astype(jnp.float32)
                + res_v[slot, :, :]
            )
            inv = lax.rsqrt(jnp.sum(y * y, axis=-1, keepdims=True) / D + EPS)
            o = y * inv * gamma_ref[:, :]
            o_v[slot, :, :] = o
            sendB[slot, :, :] = o.astype(jnp.bfloat16)
            od = pltpu.make_async_copy(
                o_v.at[slot], out_ref.at[rows, :], outA_sems.at[slot]
            )
            od.start()
            outA.append(od)
            rb = pltpu.make_async_remote_copy(
                src_ref=sendB.at[slot],
                dst_ref=recvB.at[pl.ds(c * CH, CH), :],
                send_sem=sendB_sems.at[slot],
                recv_sem=recvB_sems.at[c],
                device_id=x_nbr,
                device_id_type=pl.DeviceIdType.MESH,
            )
            rb.start()
            rdmaB.append(rb)
            if c >= LAG:
                store_other_half(c - LAG)

        for d in range(NC - LAG, NC):
            store_other_half(d)
        for c in (NC - 2, NC - 1):
            rdmaA[c].wait_send()
            rdmaB[c].wait_send()
            outA[c].wait()
            outB[c].wait()

        @functools.partial(pl.run_scoped, sem=pltpu.SemaphoreType.REGULAR)
        def _(sem):
            for nbr in (y_nbr, x_nbr):
                pl.semaphore_signal(
                    sem, inc=1, device_id=nbr,
                    device_id_type=pl.DeviceIdType.MESH,
                )
            pl.semaphore_wait(sem, 2)

    return pl.pallas_call(
        body,
        out_shape=jax.ShapeDtypeStruct((M, D), jnp.float32),
        in_specs=[
            pl.BlockSpec(memory_space=pltpu.MemorySpace.HBM),
            pl.BlockSpec(memory_space=pltpu.MemorySpace.HBM),
            pl.BlockSpec(memory_space=pltpu.MemorySpace.VMEM),
        ],
        out_specs=pl.BlockSpec(memory_space=pltpu.MemorySpace.HBM),
        scratch_shapes=[
            pltpu.MemorySpace.VMEM((2, CH, D), jnp.bfloat16),
            pltpu.MemorySpace.VMEM((HALF, D), jnp.bfloat16),
            pltpu.MemorySpace.VMEM((2, CH, D), jnp.bfloat16),
            pltpu.MemorySpace.VMEM((HALF, D), jnp.bfloat16),
            pltpu.MemorySpace.VMEM((2, CH, D), jnp.float32),
            pltpu.MemorySpace.VMEM((2, CH, D), jnp.float32),
            pltpu.MemorySpace.VMEM((2, CH, D), jnp.float32),
            pltpu.MemorySpace.VMEM((2, CH, D), jnp.float32),
            pltpu.SemaphoreType.DMA((4,)),
            pltpu.SemaphoreType.DMA((2,)),
            pltpu.SemaphoreType.DMA((NC,)),
            pltpu.SemaphoreType.DMA((2,)),
            pltpu.SemaphoreType.DMA((NC,)),
            pltpu.SemaphoreType.DMA((2,)),
            pltpu.SemaphoreType.DMA((2,)),
        ],
        compiler_params=pltpu.CompilerParams(
            collective_id=0, vmem_limit_bytes=62 * 1024 * 1024
        ),
    )(partial2d, resid, gamma2d)


# device time: 247249 ns/iter; 1.7228x vs baseline; 1.3308x over previous
import functools

import jax
import jax.numpy as jnp
from jax import lax
from jax.experimental import pallas as pl
from jax.experimental.pallas import tpu as pltpu

M = 4096
D = 4096
HALF = M // 2
CH = 128
NC = HALF // CH
EPS = 1e-6


def kernel(partial, resid, gamma):
    partial2d = partial.reshape(M, D)
    gamma2d = gamma.reshape(1, D)

    def body(
        partial_ref, resid_ref, gamma_ref, out_ref,
        sendA, recvA, sendB, recvB,
        pA, pAs, res_v, o_v, oB_v,
        loc_sems, sendA_sems, recvA_sems, sendB_sems, recvB_sems,
        outA_sems, outB_sems,
    ):
        my_x = lax.axis_index("x")
        my_y = lax.axis_index("y")
        y_nbr = (my_x, 1 - my_y)
        x_nbr = (1 - my_x, my_y)

        barrier_sem = pltpu.get_barrier_semaphore()
        for nbr in (y_nbr, x_nbr):
            pl.semaphore_signal(
                barrier_sem, inc=1, device_id=nbr,
                device_id_type=pl.DeviceIdType.MESH,
            )
        pl.semaphore_wait(barrier_sem, 2)

        half_start = my_x * HALF
        other_start = HALF - half_start

        def recvA_desc(c):
            return pltpu.make_async_remote_copy(
                src_ref=sendA.at[0],
                dst_ref=recvA.at[pl.ds(c * CH, CH), :],
                send_sem=sendA_sems.at[0],
                recv_sem=recvA_sems.at[c],
                device_id=y_nbr,
                device_id_type=pl.DeviceIdType.MESH,
            )

        def recvB_desc(c):
            return pltpu.make_async_remote_copy(
                src_ref=sendB.at[0],
                dst_ref=recvB.at[pl.ds(c * CH, CH), :],
                send_sem=sendB_sems.at[0],
                recv_sem=recvB_sems.at[c],
                device_id=x_nbr,
                device_id_type=pl.DeviceIdType.MESH,
            )

        K = 3
        LAG = 2
        rdmaA, rdmaB, outA, outB = [], [], [], []

        def stage_and_send_A(j):
            if j >= 4:
                rdmaA[j - 4].wait_send()
            cp = pltpu.make_async_copy(
                partial_ref.at[pl.ds(half_start + j * CH, CH), :],
                pAs.at[j % 2],
                loc_sems.at[0],
            )
            cp.start()
            cp.wait()
            sendA[j % 4, :, :] = pAs[j % 2, :, :].astype(jnp.bfloat16)
            r = pltpu.make_async_remote_copy(
                src_ref=sendA.at[j % 4],
                dst_ref=recvA.at[pl.ds(j * CH, CH), :],
                send_sem=sendA_sems.at[j % 4],
                recv_sem=recvA_sems.at[j],
                device_id=y_nbr,
                device_id_type=pl.DeviceIdType.MESH,
            )
            r.start()
            rdmaA.append(r)

        def store_other_half(d):
            dslot = d % 2
            orows = pl.ds(other_start + d * CH, CH)
            recvB_desc(d).wait_recv()
            if d >= 2:
                outB[d - 2].wait()
            oB_v[dslot, :, :] = recvB[d * CH:(d + 1) * CH, :].astype(
                jnp.float32
            )
            od = pltpu.make_async_copy(
                oB_v.at[dslot], out_ref.at[orows, :], outB_sems.at[dslot]
            )
            od.start()
            outB.append(od)

        for j in range(min(K, NC)):
            stage_and_send_A(j)

        for c in range(NC):
            slot = c % 2
            rows = pl.ds(half_start + c * CH, CH)
            if c >= 2:
                rdmaB[c - 2].wait_send()
                outA[c - 2].wait()
            cp1 = pltpu.make_async_copy(
                partial_ref.at[rows, :], pA.at[slot], loc_sems.at[1]
            )
            cp2 = pltpu.make_async_copy(
                resid_ref.at[rows, :], res_v.at[slot], loc_sems.at[2]
            )
            cp1.start(); cp2.start()
            if c + K < NC:
                stage_and_send_A(c + K)
            recvA_desc(c).wait_recv()
            cp1.wait()
            cp2.wait()
            y = (
                pA[slot, :, :]
                + recvA[c * CH:(c + 1) * CH, :].astype(jnp.float32)
                + res_v[slot, :, :]
            )
            inv = lax.rsqrt(jnp.sum(y * y, axis=-1, keepdims=True) / D + EPS)
            o = y * inv * gamma_ref[:, :]
            o_v[slot, :, :] = o
            sendB[slot, :, :] = o.astype(jnp.bfloat16)
            od = pltpu.make_async_copy(
                o_v.at[slot], out_ref.at[rows, :], outA_sems.at[slot]
            )
            od.start()
            outA.append(od)
            rb = pltpu.make_async_remote_copy(
                src_ref=sendB.at[slot],
                dst_ref=recvB.at[pl.ds(c * CH, CH), :],
                send_sem=sendB_sems.at[slot],
                recv_sem=recvB_sems.at[c],
                device_id=x_nbr,
                device_id_type=pl.DeviceIdType.MESH,
            )
            rb.start()
            rdmaB.append(rb)
            if c >= LAG:
                store_other_half(c - LAG)

        for d in range(NC - LAG, NC):
            store_other_half(d)
        for c in range(max(NC - 4, 0), NC):
            rdmaA[c].wait_send()
        for c in (NC - 2, NC - 1):
            rdmaB[c].wait_send()
            outA[c].wait()
            outB[c].wait()

        @functools.partial(pl.run_scoped, sem=pltpu.SemaphoreType.REGULAR)
        def _(sem):
            for nbr in (y_nbr, x_nbr):
                pl.semaphore_signal(
                    sem, inc=1, device_id=nbr,
                    device_id_type=pl.DeviceIdType.MESH,
                )
            pl.semaphore_wait(sem, 2)

    return pl.pallas_call(
        body,
        out_shape=jax.ShapeDtypeStruct((M, D), jnp.float32),
        in_specs=[
            pl.BlockSpec(memory_space=pltpu.MemorySpace.HBM),
            pl.BlockSpec(memory_space=pltpu.MemorySpace.HBM),
            pl.BlockSpec(memory_space=pltpu.MemorySpace.VMEM),
        ],
        out_specs=pl.BlockSpec(memory_space=pltpu.MemorySpace.HBM),
        scratch_shapes=[
            pltpu.MemorySpace.VMEM((4, CH, D), jnp.bfloat16),
            pltpu.MemorySpace.VMEM((HALF, D), jnp.bfloat16),
            pltpu.MemorySpace.VMEM((2, CH, D), jnp.bfloat16),
            pltpu.MemorySpace.VMEM((HALF, D), jnp.bfloat16),
            pltpu.MemorySpace.VMEM((2, CH, D), jnp.float32),
            pltpu.MemorySpace.VMEM((2, CH, D), jnp.float32),
            pltpu.MemorySpace.VMEM((2, CH, D), jnp.float32),
            pltpu.MemorySpace.VMEM((2, CH, D), jnp.float32),
            pltpu.MemorySpace.VMEM((2, CH, D), jnp.float32),
            pltpu.SemaphoreType.DMA((4,)),
            pltpu.SemaphoreType.DMA((4,)),
            pltpu.SemaphoreType.DMA((NC,)),
            pltpu.SemaphoreType.DMA((2,)),
            pltpu.SemaphoreType.DMA((NC,)),
            pltpu.SemaphoreType.DMA((2,)),
            pltpu.SemaphoreType.DMA((2,)),
        ],
        compiler_params=pltpu.CompilerParams(
            collective_id=0, vmem_limit_bytes=62 * 1024 * 1024
        ),
    )(partial2d, resid, gamma2d)
